# Initial kernel scaffold; baseline (speedup 1.0000x reference)
import jax
import jax.numpy as jnp
from jax.experimental import pallas as pl


def kernel(x, adj_indices, adj_values, W, b):
    raise NotImplementedError("write your pallas kernel here")


# speedup vs baseline: 7.5377x; 7.5377x over previous
import functools

import jax
import jax.numpy as jnp
from jax import lax
from jax.experimental import pallas as pl
from jax.experimental.pallas import tpu as pltpu
from jax.experimental.pallas import tpu_sc as plsc

N_NODES = 10000
N_EDGES = 320000
D = 128
NC = 2
NS = 16
SCH = 1024
CH = 256
JW = CH // 128
SUB = SCH // CH
ROWS_PT = 640
NPAD = NS * ROWS_PT
EP = 327680
EPT = EP // (NC * NS)
SCHUNKS = EPT // SCH
ER = EP // 128


def _mm_body(x_ref, wt_ref, b_ref, o_ref):
    o_ref[...] = (
        jnp.dot(x_ref[...], wt_ref[...], preferred_element_type=jnp.float32)
        + b_ref[...]
    )


def _support(x, wt, b2):
    return pl.pallas_call(
        _mm_body,
        grid=(5,),
        in_specs=[
            pl.BlockSpec((2000, D), lambda i: (i, 0)),
            pl.BlockSpec((D, D), lambda i: (0, 0)),
            pl.BlockSpec((1, D), lambda i: (0, 0)),
        ],
        out_specs=pl.BlockSpec((2000, D), lambda i: (i, 0)),
        out_shape=jax.ShapeDtypeStruct((N_NODES, D), jnp.float32),
    )(x, wt, b2)


_MESH = plsc.VectorSubcoreMesh(core_axis_name="c", subcore_axis_name="s")


@functools.partial(
    pl.kernel,
    mesh=_MESH,
    out_type=jax.ShapeDtypeStruct((NC, NPAD, D), jnp.float32),
    scratch_types=[
        pltpu.VMEM((SCH // 128, 128), jnp.int32),
        pltpu.VMEM((SCH // 128, 128), jnp.int32),
        pltpu.VMEM((SCH,), jnp.float32),
        pltpu.VMEM((CH, D), jnp.float32),
        pltpu.VMEM_SHARED((NPAD, D), jnp.float32),
        pltpu.SemaphoreType.DMA,
    ],
)
def _agg(sup, rows2d, cols2d, vals, out, colv, rowv, valv, gbuf, acc, sem):
    cid = lax.axis_index("c")
    sid = lax.axis_index("s")

    def _z(i, c):
        for j in range(D // 16):
            gbuf[i, pl.ds(j * 16, 16)] = jnp.zeros((16,), jnp.float32)
        return c

    lax.fori_loop(0, CH, _z, 0)
    z0 = pl.multiple_of(sid * ROWS_PT, ROWS_PT)
    pltpu.sync_copy(gbuf, acc.at[pl.ds(z0, CH)])
    pltpu.sync_copy(gbuf, acc.at[pl.ds(z0 + CH, CH)])
    pltpu.sync_copy(
        gbuf.at[pl.ds(0, ROWS_PT - 2 * CH)],
        acc.at[pl.ds(z0 + 2 * CH, ROWS_PT - 2 * CH)],
    )
    plsc.subcore_barrier()

    tile_base = pl.multiple_of((cid * NS + sid) * EPT, EPT)
    idx_base = pl.multiple_of((cid * NS + sid) * (EPT // 128), EPT // 128)

    def _schunk(k, c):
        pltpu.sync_copy(
            cols2d.at[pl.ds(idx_base + k * (SCH // 128), SCH // 128)], colv
        )
        pltpu.sync_copy(
            rows2d.at[pl.ds(idx_base + k * (SCH // 128), SCH // 128)], rowv
        )
        pltpu.sync_copy(vals.at[pl.ds(tile_base + k * SCH, SCH)], valv)
        for s in range(SUB):
            j0 = s * JW
            cps = [
                pltpu.async_copy(
                    sup.at[colv.at[j0 + j]], gbuf.at[pl.ds(j * 128, 128)], sem
                )
                for j in range(JW)
            ]
            for cp in cps:
                cp.wait()

            def _scale(g, cc, _s=s):
                vv = valv[pl.ds(_s * CH + g * 16, 16)]
                for t in range(16):
                    v = vv[t]
                    e = g * 16 + t
                    for j in range(D // 16):
                        gbuf[e, pl.ds(j * 16, 16)] = (
                            gbuf[e, pl.ds(j * 16, 16)] * v
                        )
                return cc

            lax.fori_loop(0, CH // 16, _scale, 0)
            for j in range(JW):
                pltpu.sync_copy(
                    gbuf.at[pl.ds(j * 128, 128)],
                    acc.at[rowv.at[j0 + j]],
                    add=True,
                )
        return c

    lax.fori_loop(0, SCHUNKS, _schunk, 0)

    plsc.subcore_barrier()
    pltpu.sync_copy(
        acc.at[pl.ds(sid * ROWS_PT, ROWS_PT)],
        out.at[cid, pl.ds(sid * ROWS_PT, ROWS_PT)],
    )


def _add_body(a_ref, b_ref, o_ref):
    o_ref[...] = a_ref[...] + b_ref[...]


def _combine(p0, p1):
    return pl.pallas_call(
        _add_body,
        grid=(5,),
        in_specs=[
            pl.BlockSpec((2000, D), lambda i: (i, 0)),
            pl.BlockSpec((2000, D), lambda i: (i, 0)),
        ],
        out_specs=pl.BlockSpec((2000, D), lambda i: (i, 0)),
        out_shape=jax.ShapeDtypeStruct((N_NODES, D), jnp.float32),
    )(p0, p1)


def kernel(x, adj_indices, adj_values, W, b):
    rows = adj_indices[0].astype(jnp.int32)
    cols = adj_indices[1].astype(jnp.int32)
    pad = EP - N_EDGES
    spread = (jnp.arange(pad, dtype=jnp.int32) * 37) % N_NODES
    rows_p = jnp.concatenate([rows, spread]).reshape(ER, 128)
    cols_p = jnp.concatenate([cols, spread]).reshape(ER, 128)
    vals_p = jnp.concatenate([adj_values, jnp.zeros((pad,), jnp.float32)])
    sup = _support(x, W.T, b.reshape(1, D))
    part = _agg(sup, rows_p, cols_p, vals_p)
    return _combine(part[0], part[1])
